# baseline (speedup 1.0000x reference)
import functools

import jax
import jax.numpy as jnp
from jax import lax
from jax.experimental import pallas as pl
from jax.experimental.pallas import tpu as pltpu
from jax.experimental.pallas import tpu_sc as plsc

_B, _P = 131072, 256
_L = 8
_NC, _NS, _LANES = 2, 16, 16
_NW = _NC * _NS
_ROWS_PER_W = _B // _NW
_CH = 128
_NCHUNK = _ROWS_PER_W // _CH

_mesh = plsc.VectorSubcoreMesh(core_axis_name="c", subcore_axis_name="s")


def _compute_chunk(in_v, w_vec, out_v):
    iota = lax.iota(jnp.int32, _LANES)
    for g in range(_CH // _LANES):
        rows = iota + (g * _LANES)
        vs = [in_v[pl.ds(j * _CH + g * _LANES, _LANES)] for j in range(_L)]
        sgn = [(-1.0 if j % 2 == 0 else 1.0) for j in range(_L)]
        cm = [vs[j] * sgn[j] for j in range(_L)]
        m = cm[0]
        for j in range(1, _L):
            m = jnp.maximum(m, cm[j])
        es = [jnp.exp(cm[j] - m) for j in range(_L)]
        s = es[0]
        for j in range(1, _L):
            s = s + es[j]
        scale = w_vec / s
        for j in range(_L):
            plsc.store_scatter(
                out_v, [rows * _P + j], es[j] * (scale * sgn[j]))


@functools.partial(
    pl.kernel,
    mesh=_mesh,
    compiler_params=pltpu.CompilerParams(needs_layout_passes=False),
    out_type=jax.ShapeDtypeStruct((_B * _P,), jnp.float32),
    scratch_types=(
        [pltpu.VMEM((_CH,), jnp.int32) for _ in range(_L)]
        + [
            pltpu.VMEM((_L * _CH,), jnp.float32),
            pltpu.VMEM((_L * _CH,), jnp.float32),
            pltpu.VMEM((_LANES,), jnp.float32),
            pltpu.VMEM((_CH * _P,), jnp.float32),
            pltpu.VMEM((_CH * _P,), jnp.float32),
            pltpu.SemaphoreType.DMA,
            pltpu.SemaphoreType.DMA,
            pltpu.SemaphoreType.DMA,
        ]
    ),
)
def _sc_kernel(in_hbm, w_hbm, out_hbm,
               i0, i1, i2, i3, i4, i5, i6, i7,
               in_v0, in_v1, w_v, ov0, ov1, sem0, sem1, gsem):
    idx_refs = (i0, i1, i2, i3, i4, i5, i6, i7)
    wid = lax.axis_index("s") * _NC + lax.axis_index("c")
    row0 = wid * _ROWS_PER_W

    pltpu.sync_copy(w_hbm, w_v)
    w_vec = w_v[...]

    zero = jnp.zeros((_LANES,), jnp.float32)

    def _zero_blk(t, _):
        ov0[pl.ds(t * _LANES, _LANES)] = zero
        ov1[pl.ds(t * _LANES, _LANES)] = zero
        return _

    lax.fori_loop(0, (_CH * _P) // _LANES, _zero_blk, None)

    iota = lax.iota(jnp.int32, _LANES)
    in_bufs = (in_v0, in_v1)
    out_bufs = (ov0, ov1)
    sems = (sem0, sem1)

    def _pair(i, _):
        for b in range(2):
            chunk = i * 2 + b
            base = row0 + chunk * _CH

            @pl.when(i > 0)
            def _():
                pltpu.make_async_copy(
                    out_bufs[b], out_hbm.at[pl.ds(0, _CH * _P)], sems[b]
                ).wait()

            base_off = base * _P
            for j in range(_L):
                for t in range(_CH // _LANES):
                    idx_refs[j][pl.ds(t * _LANES, _LANES)] = (
                        base_off + (iota + t * _LANES) * _P + j)
            for j in range(_L):
                pltpu.async_copy(
                    in_hbm.at[idx_refs[j]],
                    in_bufs[b].at[pl.ds(j * _CH, _CH)],
                    gsem,
                )
            for j in range(_L):
                pltpu.make_async_copy(
                    in_hbm.at[idx_refs[j]],
                    in_bufs[b].at[pl.ds(j * _CH, _CH)],
                    gsem,
                ).wait()

            _compute_chunk(in_bufs[b], w_vec, out_bufs[b])
            pltpu.async_copy(
                out_bufs[b], out_hbm.at[pl.ds(base * _P, _CH * _P)], sems[b])
        return _

    lax.fori_loop(0, _NCHUNK // 2, _pair, None)
    for b in range(2):
        pltpu.make_async_copy(
            out_bufs[b], out_hbm.at[pl.ds(0, _CH * _P)], sems[b]
        ).wait()


@jax.jit
def kernel(inputs, clause_weight):
    w16 = jnp.broadcast_to(clause_weight.reshape(()), (_LANES,))
    out = _sc_kernel(inputs.reshape(-1), w16)
    return out.reshape(_B, _P)


# speedup vs baseline: 3.8605x; 3.8605x over previous
import functools

import jax
import jax.numpy as jnp
from jax import lax
from jax.experimental import pallas as pl
from jax.experimental.pallas import tpu as pltpu
from jax.experimental.pallas import tpu_sc as plsc

_B, _P = 131072, 256
_L = 8
_NC, _NS, _LANES = 2, 16, 16
_NW = _NC * _NS
_ROWS_PER_W = _B // _NW
_CH = 128
_NB = _CH // 8
_NCHUNK = _ROWS_PER_W // _CH

_mesh = plsc.VectorSubcoreMesh(core_axis_name="c", subcore_axis_name="s")


def _compute_chunk(in_v, w_vec, out_v):
    iota = lax.iota(jnp.int32, _LANES)
    for g in range(_CH // _LANES):
        rows = iota + (g * _LANES)
        band = rows // 8
        sub = rows % 8
        cols = [jnp.full((_LANES,), j, jnp.int32) for j in range(_L)]
        vs = [plsc.load_gather(in_v, [band, sub, cols[j]]) for j in range(_L)]
        sgn = [(-1.0 if j % 2 == 0 else 1.0) for j in range(_L)]
        cm = [vs[j] * sgn[j] for j in range(_L)]
        m = cm[0]
        for j in range(1, _L):
            m = jnp.maximum(m, cm[j])
        es = [jnp.exp(cm[j] - m) for j in range(_L)]
        s = es[0]
        for j in range(1, _L):
            s = s + es[j]
        scale = w_vec / s
        for j in range(_L):
            plsc.store_scatter(out_v, [rows, cols[j]], es[j] * (scale * sgn[j]))


@functools.partial(
    pl.kernel,
    mesh=_mesh,
    compiler_params=pltpu.CompilerParams(needs_layout_passes=False),
    out_type=jax.ShapeDtypeStruct((_B, _P), jnp.float32),
    scratch_types=[
        pltpu.VMEM((_NB, 8, 128), jnp.float32),
        pltpu.VMEM((_NB, 8, 128), jnp.float32),
        pltpu.VMEM((_LANES,), jnp.float32),
        pltpu.VMEM((_CH, _P), jnp.float32),
        pltpu.VMEM((_CH, _P), jnp.float32),
        pltpu.SemaphoreType.DMA,
        pltpu.SemaphoreType.DMA,
        pltpu.SemaphoreType.DMA,
        pltpu.SemaphoreType.DMA,
    ],
)
def _sc_kernel(in3_hbm, w_hbm, out_hbm,
               in_v0, in_v1, w_v, ov0, ov1, osem0, osem1, isem0, isem1):
    wid = lax.axis_index("s") * _NC + lax.axis_index("c")
    row0 = wid * _ROWS_PER_W

    pltpu.sync_copy(w_hbm, w_v)
    w_vec = w_v[...]

    zero = jnp.zeros((_LANES,), jnp.float32)

    def _zero_row(r, _):
        for c in range(_P // _LANES):
            ov0[r, pl.ds(c * _LANES, _LANES)] = zero
            ov1[r, pl.ds(c * _LANES, _LANES)] = zero
        return _

    lax.fori_loop(0, _CH, _zero_row, None)

    in_bufs = (in_v0, in_v1)
    out_bufs = (ov0, ov1)
    osems = (osem0, osem1)
    isems = (isem0, isem1)

    def _fetch(chunk, b):
        band0 = (row0 + chunk * _CH) // 8
        pltpu.async_copy(
            in3_hbm.at[pl.ds(band0, _NB), :, pl.ds(0, 128)],
            in_bufs[b], isems[b])

    def _fetch_wait(chunk, b):
        band0 = (row0 + chunk * _CH) // 8
        pltpu.make_async_copy(
            in3_hbm.at[pl.ds(band0, _NB), :, pl.ds(0, 128)],
            in_bufs[b], isems[b]).wait()

    _fetch(0, 0)

    def _pair(i, _):
        for b in range(2):
            chunk = i * 2 + b
            base = row0 + chunk * _CH
            nxt_b = 1 - b

            @pl.when(chunk + 1 < _NCHUNK)
            def _():
                _fetch(chunk + 1, nxt_b)

            _fetch_wait(chunk, b)

            @pl.when(i > 0)
            def _():
                pltpu.make_async_copy(
                    out_bufs[b], out_hbm.at[pl.ds(0, _CH)], osems[b]).wait()

            _compute_chunk(in_bufs[b], w_vec, out_bufs[b])
            pltpu.async_copy(
                out_bufs[b], out_hbm.at[pl.ds(base, _CH)], osems[b])
        return _

    lax.fori_loop(0, _NCHUNK // 2, _pair, None)
    for b in range(2):
        pltpu.make_async_copy(
            out_bufs[b], out_hbm.at[pl.ds(0, _CH)], osems[b]).wait()


@jax.jit
def kernel(inputs, clause_weight):
    in3 = inputs.reshape(_B // 8, 8, _P)
    w16 = jnp.broadcast_to(clause_weight.reshape(()), (_LANES,))
    return _sc_kernel(in3, w16)
